# baseline (device time: 42735 ns/iter reference)
import jax
import jax.numpy as jnp
from jax import lax
from jax.experimental import pallas as pl
from jax.experimental.pallas import tpu as pltpu

N_DEV = 8
HPS = 8
DH = 128
SQ = 256
SKV = 4096
DM = 1024
QB = 64
N_QB = 4
KSEL = 1024
NKB = 16
SCALE = 0.08838834764831843
STEPS = 3


def _body(x_ref, wq_ref, k_hbm, v_hbm, wo_ref, out_ref,
          stage_ref, kqb_ref, vqb_ref, ctxc_ref, wob_ref, acc_ref,
          send_ref, recv_ref, copy_sems, send_sems, recv_sems):
    my = lax.axis_index("i")
    h0 = my * HPS

    def ex_start(c, s):
        peer = my ^ (1 << s)
        send_ref[c, s, :, :] = acc_ref[c * QB:(c + 1) * QB, :].astype(
            jnp.bfloat16)
        rdma = pltpu.make_async_remote_copy(
            src_ref=send_ref.at[c, s],
            dst_ref=recv_ref.at[c, s],
            send_sem=send_sems.at[c, s],
            recv_sem=recv_sems.at[c, s],
            device_id=(peer,),
            device_id_type=pl.DeviceIdType.MESH,
        )
        rdma.start()
        return rdma

    def ex_finish(rdma, c, s):
        rdma.wait()
        acc_ref[c * QB:(c + 1) * QB, :] = (
            acc_ref[c * QB:(c + 1) * QB, :]
            + recv_ref[c, s].astype(jnp.float32))

    pend = {}

    def run(actions):
        for op, c, s in actions:
            if op == "start":
                pend[(c, s)] = ex_start(c, s)
            else:
                ex_finish(pend.pop((c, s)), c, s)

    after_attn = {
        0: [("start", 0, 0)],
        1: [("fin", 0, 0), ("start", 0, 1), ("start", 1, 0)],
        2: [("fin", 0, 1), ("start", 0, 2),
            ("fin", 1, 0), ("start", 1, 1), ("start", 2, 0)],
        3: [("fin", 0, 2),
            ("fin", 1, 1), ("start", 1, 2),
            ("fin", 2, 0), ("start", 2, 1), ("start", 3, 0)],
    }
    drain = [("fin", 1, 2),
             ("fin", 2, 1), ("start", 2, 2),
             ("fin", 3, 0), ("start", 3, 1),
             ("fin", 2, 2),
             ("fin", 3, 1), ("start", 3, 2),
             ("fin", 3, 2)]

    def start_unit(hbm, qb, slot):
        cs = []
        for h in range(HPS):
            c = pltpu.make_async_copy(
                hbm.at[:, qb, :, h0 + h, :],
                stage_ref.at[slot, h],
                copy_sems.at[slot, h])
            c.start()
            cs.append(c)
        return cs

    units = []
    for qb in range(N_QB):
        units.append((k_hbm, kqb_ref, qb))
        units.append((v_hbm, vqb_ref, qb))

    inflight = start_unit(units[0][0], units[0][2], 0)

    barrier_sem = pltpu.get_barrier_semaphore()
    for s in range(STEPS):
        peer = my ^ (1 << s)
        pl.semaphore_signal(barrier_sem, inc=1, device_id=(peer,),
                            device_id_type=pl.DeviceIdType.MESH)
    pl.semaphore_wait(barrier_sem, STEPS)

    wob_ref[...] = wo_ref[...].astype(jnp.bfloat16)
    q = (jnp.dot(x_ref[0].astype(jnp.bfloat16),
                 wq_ref[...].astype(jnp.bfloat16),
                 preferred_element_type=jnp.float32)
         * SCALE).astype(jnp.bfloat16)

    for u, (hbm, dst, qb) in enumerate(units):
        slot = u % 2
        nxt = None
        if u + 1 < len(units):
            nhbm, _, nqb = units[u + 1]
            nxt = start_unit(nhbm, nqb, (u + 1) % 2)
        for c in inflight:
            c.wait()
        for h in range(HPS):
            dst[h, :, :] = stage_ref[slot, h].reshape(
                KSEL, DH).astype(jnp.bfloat16)
        inflight = nxt

        if dst is vqb_ref:
            for h in range(HPS):
                qh = q[qb * QB:(qb + 1) * QB, h * DH:(h + 1) * DH]
                sc = lax.dot_general(qh, kqb_ref[h],
                                     (((1,), (1,)), ((), ())),
                                     preferred_element_type=jnp.float32)
                p = jnp.exp(sc)
                denom = jnp.sum(p, axis=1, keepdims=True)
                ctx_h = jnp.dot(p.astype(jnp.bfloat16), vqb_ref[h],
                                preferred_element_type=jnp.float32) / denom
                ctxc_ref[:, h * DH:(h + 1) * DH] = ctx_h.astype(jnp.bfloat16)
            acc_ref[qb * QB:(qb + 1) * QB, :] = jnp.dot(
                ctxc_ref[...], wob_ref[...],
                preferred_element_type=jnp.float32)
            run(after_attn[qb])

    run(drain)
    out_ref[0, :, :] = acc_ref[...]


def kernel(x, Wq, K_ext, V_ext, Wo):
    kr = K_ext.reshape(NKB, N_QB, QB, 64, DH)
    vr = V_ext.reshape(NKB, N_QB, QB, 64, DH)
    return pl.pallas_call(
        _body,
        out_shape=jax.ShapeDtypeStruct((1, SQ, DM), jnp.float32),
        in_specs=[
            pl.BlockSpec(memory_space=pltpu.VMEM),
            pl.BlockSpec(memory_space=pltpu.VMEM),
            pl.BlockSpec(memory_space=pl.ANY),
            pl.BlockSpec(memory_space=pl.ANY),
            pl.BlockSpec(memory_space=pltpu.VMEM),
        ],
        out_specs=pl.BlockSpec(memory_space=pltpu.VMEM),
        scratch_shapes=[
            pltpu.VMEM((2, HPS, NKB, QB, DH), jnp.float32),
            pltpu.VMEM((HPS, KSEL, DH), jnp.bfloat16),
            pltpu.VMEM((HPS, KSEL, DH), jnp.bfloat16),
            pltpu.VMEM((QB, DM), jnp.bfloat16),
            pltpu.VMEM((DM, DM), jnp.bfloat16),
            pltpu.VMEM((SQ, DM), jnp.float32),
            pltpu.VMEM((N_QB, STEPS, QB, DM), jnp.bfloat16),
            pltpu.VMEM((N_QB, STEPS, QB, DM), jnp.bfloat16),
            pltpu.SemaphoreType.DMA((2, HPS)),
            pltpu.SemaphoreType.DMA((N_QB, STEPS)),
            pltpu.SemaphoreType.DMA((N_QB, STEPS)),
        ],
        compiler_params=pltpu.CompilerParams(collective_id=0),
    )(x, Wq, kr, vr, Wo)


# device time: 40090 ns/iter; 1.0660x vs baseline; 1.0660x over previous
import jax
import jax.numpy as jnp
from jax import lax
from jax.experimental import pallas as pl
from jax.experimental.pallas import tpu as pltpu

N_DEV = 8
HPS = 8
DH = 128
SQ = 256
SKV = 4096
DM = 1024
QB = 64
N_QB = 4
KSEL = 1024
NKB = 16
SCALE = 0.08838834764831843
STEPS = 3


def _body(x_ref, wq_ref, k_hbm, v_hbm, wo_ref, out_ref,
          stage_ref, ctxc_ref, wob_ref, acc_ref,
          send_ref, recv_ref, copy_sems, send_sems, recv_sems):
    my = lax.axis_index("i")
    h0 = my * HPS

    def ex_start(c, s):
        peer = my ^ (1 << s)
        send_ref[c, s, :, :] = acc_ref[c * QB:(c + 1) * QB, :].astype(
            jnp.bfloat16)
        rdma = pltpu.make_async_remote_copy(
            src_ref=send_ref.at[c, s],
            dst_ref=recv_ref.at[c, s],
            send_sem=send_sems.at[c, s],
            recv_sem=recv_sems.at[c, s],
            device_id=(peer,),
            device_id_type=pl.DeviceIdType.MESH,
        )
        rdma.start()
        return rdma

    def ex_finish(rdma, c, s):
        rdma.wait()
        acc_ref[c * QB:(c + 1) * QB, :] = (
            acc_ref[c * QB:(c + 1) * QB, :]
            + recv_ref[c, s].astype(jnp.float32))

    pend = {}

    def run(actions):
        for op, c, s in actions:
            if op == "start":
                pend[(c, s)] = ex_start(c, s)
            else:
                ex_finish(pend.pop((c, s)), c, s)

    after_attn = {
        0: [("start", 0, 0)],
        1: [("fin", 0, 0), ("start", 0, 1), ("start", 1, 0)],
        2: [("fin", 0, 1), ("start", 0, 2),
            ("fin", 1, 0), ("start", 1, 1), ("start", 2, 0)],
        3: [("fin", 0, 2),
            ("fin", 1, 1), ("start", 1, 2),
            ("fin", 2, 0), ("start", 2, 1), ("start", 3, 0)],
    }
    drain = [("fin", 1, 2),
             ("fin", 2, 1), ("start", 2, 2),
             ("fin", 3, 0), ("start", 3, 1),
             ("fin", 2, 2),
             ("fin", 3, 1), ("start", 3, 2),
             ("fin", 3, 2)]

    def start_unit(hbm, qb, slot):
        cs = []
        for h in range(HPS):
            c = pltpu.make_async_copy(
                hbm.at[:, qb, :, h0 + h, :],
                stage_ref.at[slot, h],
                copy_sems.at[slot, h])
            c.start()
            cs.append(c)
        return cs

    units = []
    for qb in range(N_QB):
        units.append((k_hbm, qb, False))
        units.append((v_hbm, qb, True))

    inflight = [start_unit(units[0][0], units[0][1], 0),
                start_unit(units[1][0], units[1][1], 1)]

    barrier_sem = pltpu.get_barrier_semaphore()
    for s in range(STEPS):
        peer = my ^ (1 << s)
        pl.semaphore_signal(barrier_sem, inc=1, device_id=(peer,),
                            device_id_type=pl.DeviceIdType.MESH)
    pl.semaphore_wait(barrier_sem, STEPS)

    wob_ref[...] = wo_ref[...].astype(jnp.bfloat16)
    q = (jnp.dot(x_ref[0].astype(jnp.bfloat16),
                 wq_ref[...].astype(jnp.bfloat16),
                 preferred_element_type=jnp.float32)
         * SCALE).astype(jnp.bfloat16)

    for u, (hbm, qb, is_v) in enumerate(units):
        if u + 2 < len(units):
            nhbm, nqb, _ = units[u + 2]
            inflight.append(start_unit(nhbm, nqb, (u + 2) % 4))
        for c in inflight.pop(0):
            c.wait()

        if is_v:
            k_slot = (2 * qb) % 4
            v_slot = (2 * qb + 1) % 4
            for h in range(HPS):
                qh = q[qb * QB:(qb + 1) * QB, h * DH:(h + 1) * DH]
                kh = stage_ref[k_slot, h].reshape(KSEL, DH).astype(
                    jnp.bfloat16)
                sc = lax.dot_general(qh, kh,
                                     (((1,), (1,)), ((), ())),
                                     preferred_element_type=jnp.float32)
                p = jnp.exp(sc)
                denom = jnp.sum(p, axis=1, keepdims=True)
                vh = stage_ref[v_slot, h].reshape(KSEL, DH).astype(
                    jnp.bfloat16)
                ctx_h = jnp.dot(p.astype(jnp.bfloat16), vh,
                                preferred_element_type=jnp.float32) / denom
                ctxc_ref[:, h * DH:(h + 1) * DH] = ctx_h.astype(jnp.bfloat16)
            acc_ref[qb * QB:(qb + 1) * QB, :] = jnp.dot(
                ctxc_ref[...], wob_ref[...],
                preferred_element_type=jnp.float32)
            run(after_attn[qb])

    run(drain)
    out_ref[0, :, :] = acc_ref[...]


def kernel(x, Wq, K_ext, V_ext, Wo):
    kr = K_ext.reshape(NKB, N_QB, QB, 64, DH)
    vr = V_ext.reshape(NKB, N_QB, QB, 64, DH)
    return pl.pallas_call(
        _body,
        out_shape=jax.ShapeDtypeStruct((1, SQ, DM), jnp.float32),
        in_specs=[
            pl.BlockSpec(memory_space=pltpu.VMEM),
            pl.BlockSpec(memory_space=pltpu.VMEM),
            pl.BlockSpec(memory_space=pl.ANY),
            pl.BlockSpec(memory_space=pl.ANY),
            pl.BlockSpec(memory_space=pltpu.VMEM),
        ],
        out_specs=pl.BlockSpec(memory_space=pltpu.VMEM),
        scratch_shapes=[
            pltpu.VMEM((4, HPS, NKB, QB, DH), jnp.float32),
            pltpu.VMEM((QB, DM), jnp.bfloat16),
            pltpu.VMEM((DM, DM), jnp.bfloat16),
            pltpu.VMEM((SQ, DM), jnp.float32),
            pltpu.VMEM((N_QB, STEPS, QB, DM), jnp.bfloat16),
            pltpu.VMEM((N_QB, STEPS, QB, DM), jnp.bfloat16),
            pltpu.SemaphoreType.DMA((4, HPS)),
            pltpu.SemaphoreType.DMA((N_QB, STEPS)),
            pltpu.SemaphoreType.DMA((N_QB, STEPS)),
        ],
        compiler_params=pltpu.CompilerParams(collective_id=0),
    )(x, Wq, kr, vr, Wo)
